# baseline (speedup 1.0000x reference)
import functools

import jax
import jax.numpy as jnp
from jax import lax
from jax.experimental import pallas as pl
from jax.experimental.pallas import tpu as pltpu
from jax.experimental.pallas import tpu_sc as plsc

_NC = 2
_NS = 16
_L = 16

_EC = 80
_CB = 25
_NB = 5


@functools.lru_cache(maxsize=None)
def _make_aggregate(N, NP, E, Dh):
    n_per_tile = NP // _NS
    e_per_tile = E // _NS
    n_chunks_init = n_per_tile // _EC
    assert n_per_tile % _EC == 0
    assert e_per_tile == _NB * _CB * _EC and _CB % 2 == 1

    mesh = plsc.VectorSubcoreMesh(core_axis_name="c", subcore_axis_name="s")

    def body(xpk_hbm, ea2_hbm, sd_hbm, out_hbm,
             haggr, bufb0, bufb1, bufe0, bufe1,
             idxx0, idxx1, idxe0, idxe1, dstb0, dstb1,
             sd_b,
             semx0, semx1, seme0, seme1):
        k = lax.axis_index("c")
        t = lax.axis_index("s")
        lanes = jnp.arange(_L, dtype=jnp.int32)
        W = 2 * _L

        def zrows(i, _):
            for rr in range(4):
                r = i * 4 + rr
                for j in range(Dh // _L):
                    bufe0[r, pl.ds(j * _L, _L)] = jnp.zeros((_L,), jnp.float32)
            return 0

        lax.fori_loop(0, _EC // 4, zrows, 0)

        def init_chunk(c, _):
            base = t * n_per_tile + c * _EC
            pltpu.sync_copy(bufe0, haggr.at[pl.ds(base, _EC)])
            return 0

        lax.fori_loop(0, n_chunks_init, init_chunk, 0)
        plsc.subcore_barrier()

        def build_idx(lc, idxx, dstb):
            for i in range(_EC // _L):
                sl = pl.ds(i * _L, _L)
                idxx[sl] = sd_b[0, pl.ds(lc * _EC + i * _L, _L)] * 2 + k
                dstb[sl] = sd_b[1, pl.ds(lc * _EC + i * _L, _L)]

        def issue_gather(e0, idxx, idxe, bufb, bufe, semx, seme):
            pltpu.async_copy(xpk_hbm.at[idxx], bufb, semx)
            for i in range(_EC // _L):
                idxe[pl.ds(i * _L, _L)] = (e0 + i * _L + lanes) * 2 + k
            pltpu.async_copy(ea2_hbm.at[idxe], bufe, seme)

        def wait_gather(idxx, idxe, bufb, bufe, semx, seme):
            pltpu.make_async_copy(xpk_hbm.at[idxx], bufb, semx).wait()
            pltpu.make_async_copy(ea2_hbm.at[idxe], bufe, seme).wait()

        def compute(bufb, bufe):
            def rows(i, _):
                for rr in range(4):
                    r = i * 4 + rr
                    for j in range(Dh // W):
                        w = bufb[r, pl.ds(j * _L, _L)]
                        xa = lax.bitcast_convert_type(w << 16, jnp.float32)
                        xb = lax.bitcast_convert_type(w & jnp.int32(-65536), jnp.float32)
                        sa = pl.ds(j * W, _L)
                        sb = pl.ds(j * W + _L, _L)
                        bufe[r, sa] = jnp.maximum(xa + bufe[r, sa], 0.0)
                        bufe[r, sb] = jnp.maximum(xb + bufe[r, sb], 0.0)
                return 0

            lax.fori_loop(0, _EC // 4, rows, 0)

        def scatter(bufe, dstb):
            pltpu.sync_copy(bufe, haggr.at[dstb], add=True)

        def batch(b, _):
            pltpu.sync_copy(sd_hbm.at[t, b], sd_b)
            e_base = (t * _NB + b) * _CB * _EC

            build_idx(0, idxx0, dstb0)
            issue_gather(e_base, idxx0, idxe0, bufb0, bufe0, semx0, seme0)

            def pair(j, _):
                c0 = j * 2
                e1 = e_base + (c0 + 1) * _EC
                e2 = e_base + (c0 + 2) * _EC
                build_idx(c0 + 1, idxx1, dstb1)
                issue_gather(e1, idxx1, idxe1, bufb1, bufe1, semx1, seme1)
                wait_gather(idxx0, idxe0, bufb0, bufe0, semx0, seme0)
                compute(bufb0, bufe0)
                scatter(bufe0, dstb0)
                build_idx(c0 + 2, idxx0, dstb0)
                issue_gather(e2, idxx0, idxe0, bufb0, bufe0, semx0, seme0)
                wait_gather(idxx1, idxe1, bufb1, bufe1, semx1, seme1)
                compute(bufb1, bufe1)
                scatter(bufe1, dstb1)
                return 0

            lax.fori_loop(0, (_CB - 1) // 2, pair, 0)

            wait_gather(idxx0, idxe0, bufb0, bufe0, semx0, seme0)
            compute(bufb0, bufe0)
            scatter(bufe0, dstb0)
            return 0

        lax.fori_loop(0, _NB, batch, 0)
        plsc.subcore_barrier()

        base = t * n_per_tile
        pltpu.sync_copy(haggr.at[pl.ds(base, n_per_tile)],
                        out_hbm.at[k, pl.ds(base, n_per_tile)])

    return pl.kernel(
        body,
        out_type=jax.ShapeDtypeStruct((_NC, NP, Dh), jnp.float32),
        mesh=mesh,
        compiler_params=pltpu.CompilerParams(use_tc_tiling_on_sc=False),
        scratch_types=[
            pltpu.VMEM_SHARED((NP, Dh), jnp.float32),
            pltpu.VMEM((_EC, Dh // 2), jnp.int32),
            pltpu.VMEM((_EC, Dh // 2), jnp.int32),
            pltpu.VMEM((_EC, Dh), jnp.float32),
            pltpu.VMEM((_EC, Dh), jnp.float32),
            pltpu.VMEM((_EC,), jnp.int32),
            pltpu.VMEM((_EC,), jnp.int32),
            pltpu.VMEM((_EC,), jnp.int32),
            pltpu.VMEM((_EC,), jnp.int32),
            pltpu.VMEM((_EC,), jnp.int32),
            pltpu.VMEM((_EC,), jnp.int32),
            pltpu.VMEM((2, _CB * _EC), jnp.int32),
            pltpu.SemaphoreType.DMA,
            pltpu.SemaphoreType.DMA,
            pltpu.SemaphoreType.DMA,
            pltpu.SemaphoreType.DMA,
        ],
    )


def _mlp_body(x0_ref, x1_ref, h0_ref, h1_ref, w1a_ref, w1b_ref, b1_ref,
              w2_ref, b2_ref, out_ref):
    a0 = x0_ref[...] + h0_ref[0]
    a1 = x1_ref[...] + h1_ref[0]
    tm = jnp.dot(a0, w1a_ref[...], preferred_element_type=jnp.float32)
    tm = tm + jnp.dot(a1, w1b_ref[...], preferred_element_type=jnp.float32)
    tm = jnp.maximum(tm + b1_ref[...], 0.0)
    out_ref[...] = jnp.dot(tm, w2_ref[...], preferred_element_type=jnp.float32) + b2_ref[...]


@functools.lru_cache(maxsize=None)
def _make_mlp(N, NP, D, R=1000):
    Dh = D // 2
    grid = (N // R,)
    return pl.pallas_call(
        _mlp_body,
        grid=grid,
        in_specs=[
            pl.BlockSpec((R, Dh), lambda i: (i, 0)),
            pl.BlockSpec((R, Dh), lambda i: (i, 1)),
            pl.BlockSpec((1, R, Dh), lambda i: (0, i, 0)),
            pl.BlockSpec((1, R, Dh), lambda i: (1, i, 0)),
            pl.BlockSpec((Dh, D), lambda i: (0, 0)),
            pl.BlockSpec((Dh, D), lambda i: (1, 0)),
            pl.BlockSpec((1, D), lambda i: (0, 0)),
            pl.BlockSpec((D, D), lambda i: (0, 0)),
            pl.BlockSpec((1, D), lambda i: (0, 0)),
        ],
        out_specs=pl.BlockSpec((R, D), lambda i: (i, 0)),
        out_shape=jax.ShapeDtypeStruct((N, D), jnp.float32),
    )


def kernel(x, edge_index, edge_attr, W1, b1, W2, b2):
    N, D = x.shape
    E = edge_attr.shape[0]
    Dh = D // 2
    NP = ((N + _NS * _EC - 1) // (_NS * _EC)) * (_NS * _EC)
    xb = x.astype(jnp.bfloat16).reshape(N * 2, Dh // 32, 2, 16)
    xb = xb.transpose(0, 1, 3, 2)
    xpk = jax.lax.bitcast_convert_type(xb, jnp.int32).reshape(N * 2, Dh // 2)
    ea2 = edge_attr.reshape(E * 2, Dh)
    sd = edge_index.reshape(2, _NS, _NB, _CB * _EC).transpose(1, 2, 0, 3)
    h2 = _make_aggregate(N, NP, E, Dh)(xpk, ea2, sd)
    out = _make_mlp(N, NP, D)(x, x, h2, h2, W1, W1, b1.reshape(1, D), W2,
                              b2.reshape(1, D))
    return out


# speedup vs baseline: 2.0246x; 2.0246x over previous
import functools

import jax
import jax.numpy as jnp
from jax import lax
from jax.experimental import pallas as pl
from jax.experimental.pallas import tpu as pltpu
from jax.experimental.pallas import tpu_sc as plsc

_NC = 2
_NS = 16
_L = 16

_EC = 80
_CB = 25
_NB = 5


@functools.lru_cache(maxsize=None)
def _make_aggregate(N, NP, E, Dh):
    n_per_tile = NP // _NS
    e_per_tile = E // _NS
    n_chunks_init = n_per_tile // _EC
    assert n_per_tile % _EC == 0
    assert e_per_tile == _NB * _CB * _EC and _CB % 2 == 1

    mesh = plsc.VectorSubcoreMesh(core_axis_name="c", subcore_axis_name="s")

    def body(xpk_hbm, ea2_hbm, sd_hbm, out_hbm,
             haggr, bufb0, bufb1, bufe0, bufe1,
             idxx0, idxx1, idxe0, idxe1, dstb0, dstb1,
             sd_b,
             semx0, semx1, seme0, seme1):
        k = lax.axis_index("c")
        t = lax.axis_index("s")
        lanes = jnp.arange(_L, dtype=jnp.int32)
        W = 2 * _L

        def zrows(i, _):
            for rr in range(4):
                r = i * 4 + rr
                for j in range(Dh // _L):
                    bufe0[r, pl.ds(j * _L, _L)] = jnp.zeros((_L,), jnp.float32)
            return 0

        lax.fori_loop(0, _EC // 4, zrows, 0)

        def init_chunk(c, _):
            base = t * n_per_tile + c * _EC
            pltpu.sync_copy(bufe0, haggr.at[pl.ds(base, _EC)])
            return 0

        lax.fori_loop(0, n_chunks_init, init_chunk, 0)
        plsc.subcore_barrier()

        def build_idx(lc, idxx, dstb):
            for i in range(_EC // _L):
                sl = pl.ds(i * _L, _L)
                idxx[sl] = sd_b[0, pl.ds(lc * _EC + i * _L, _L)] * 2 + k
                dstb[sl] = sd_b[1, pl.ds(lc * _EC + i * _L, _L)]

        def issue_gather(e0, idxx, idxe, bufb, bufe, semx, seme):
            pltpu.async_copy(xpk_hbm.at[idxx], bufb, semx)
            for i in range(_EC // _L):
                idxe[pl.ds(i * _L, _L)] = (e0 + i * _L + lanes) * 2 + k
            pltpu.async_copy(ea2_hbm.at[idxe], bufe, seme)

        def wait_gather(idxx, idxe, bufb, bufe, semx, seme):
            pltpu.make_async_copy(xpk_hbm.at[idxx], bufb, semx).wait()
            pltpu.make_async_copy(ea2_hbm.at[idxe], bufe, seme).wait()

        def compute(bufb, bufe):
            def rows(i, _):
                for rr in range(4):
                    r = i * 4 + rr
                    for j in range(Dh // _L):
                        sl = pl.ds(j * _L, _L)
                        bufe[r, sl] = jnp.maximum(bufb[r, sl] + bufe[r, sl], 0.0)
                return 0

            lax.fori_loop(0, _EC // 4, rows, 0)

        def scatter(bufe, dstb):
            pltpu.sync_copy(bufe, haggr.at[dstb], add=True)

        def batch(b, _):
            pltpu.sync_copy(sd_hbm.at[t, b], sd_b)
            e_base = (t * _NB + b) * _CB * _EC

            build_idx(0, idxx0, dstb0)
            issue_gather(e_base, idxx0, idxe0, bufb0, bufe0, semx0, seme0)

            def pair(j, _):
                c0 = j * 2
                e1 = e_base + (c0 + 1) * _EC
                e2 = e_base + (c0 + 2) * _EC
                build_idx(c0 + 1, idxx1, dstb1)
                issue_gather(e1, idxx1, idxe1, bufb1, bufe1, semx1, seme1)
                wait_gather(idxx0, idxe0, bufb0, bufe0, semx0, seme0)
                compute(bufb0, bufe0)
                scatter(bufe0, dstb0)
                build_idx(c0 + 2, idxx0, dstb0)
                issue_gather(e2, idxx0, idxe0, bufb0, bufe0, semx0, seme0)
                wait_gather(idxx1, idxe1, bufb1, bufe1, semx1, seme1)
                compute(bufb1, bufe1)
                scatter(bufe1, dstb1)
                return 0

            lax.fori_loop(0, (_CB - 1) // 2, pair, 0)

            wait_gather(idxx0, idxe0, bufb0, bufe0, semx0, seme0)
            compute(bufb0, bufe0)
            scatter(bufe0, dstb0)
            return 0

        lax.fori_loop(0, _NB, batch, 0)
        plsc.subcore_barrier()

        base = t * n_per_tile
        pltpu.sync_copy(haggr.at[pl.ds(base, n_per_tile)],
                        out_hbm.at[k, pl.ds(base, n_per_tile)])

    return pl.kernel(
        body,
        out_type=jax.ShapeDtypeStruct((_NC, NP, Dh), jnp.float32),
        mesh=mesh,
        scratch_types=[
            pltpu.VMEM_SHARED((NP, Dh), jnp.float32),
            pltpu.VMEM((_EC, Dh), jnp.float32),
            pltpu.VMEM((_EC, Dh), jnp.float32),
            pltpu.VMEM((_EC, Dh), jnp.float32),
            pltpu.VMEM((_EC, Dh), jnp.float32),
            pltpu.VMEM((_EC,), jnp.int32),
            pltpu.VMEM((_EC,), jnp.int32),
            pltpu.VMEM((_EC,), jnp.int32),
            pltpu.VMEM((_EC,), jnp.int32),
            pltpu.VMEM((_EC,), jnp.int32),
            pltpu.VMEM((_EC,), jnp.int32),
            pltpu.VMEM((2, _CB * _EC), jnp.int32),
            pltpu.SemaphoreType.DMA,
            pltpu.SemaphoreType.DMA,
            pltpu.SemaphoreType.DMA,
            pltpu.SemaphoreType.DMA,
        ],
    )


def _mlp_body(x0_ref, x1_ref, h0_ref, h1_ref, w1a_ref, w1b_ref, b1_ref,
              w2_ref, b2_ref, out_ref):
    a0 = x0_ref[...] + h0_ref[0]
    a1 = x1_ref[...] + h1_ref[0]
    tm = jnp.dot(a0, w1a_ref[...], preferred_element_type=jnp.float32)
    tm = tm + jnp.dot(a1, w1b_ref[...], preferred_element_type=jnp.float32)
    tm = jnp.maximum(tm + b1_ref[...], 0.0)
    out_ref[...] = jnp.dot(tm, w2_ref[...], preferred_element_type=jnp.float32) + b2_ref[...]


@functools.lru_cache(maxsize=None)
def _make_mlp(N, NP, D, R=1000):
    Dh = D // 2
    grid = (N // R,)
    return pl.pallas_call(
        _mlp_body,
        grid=grid,
        in_specs=[
            pl.BlockSpec((R, Dh), lambda i: (i, 0)),
            pl.BlockSpec((R, Dh), lambda i: (i, 1)),
            pl.BlockSpec((1, R, Dh), lambda i: (0, i, 0)),
            pl.BlockSpec((1, R, Dh), lambda i: (1, i, 0)),
            pl.BlockSpec((Dh, D), lambda i: (0, 0)),
            pl.BlockSpec((Dh, D), lambda i: (1, 0)),
            pl.BlockSpec((1, D), lambda i: (0, 0)),
            pl.BlockSpec((D, D), lambda i: (0, 0)),
            pl.BlockSpec((1, D), lambda i: (0, 0)),
        ],
        out_specs=pl.BlockSpec((R, D), lambda i: (i, 0)),
        out_shape=jax.ShapeDtypeStruct((N, D), jnp.float32),
    )


def kernel(x, edge_index, edge_attr, W1, b1, W2, b2):
    N, D = x.shape
    E = edge_attr.shape[0]
    Dh = D // 2
    NP = ((N + _NS * _EC - 1) // (_NS * _EC)) * (_NS * _EC)
    x2 = x.reshape(N * 2, Dh)
    ea2 = edge_attr.reshape(E * 2, Dh)
    sd = edge_index.reshape(2, _NS, _NB, _CB * _EC).transpose(1, 2, 0, 3)
    h2 = _make_aggregate(N, NP, E, Dh)(x2, ea2, sd)
    out = _make_mlp(N, NP, D)(x, x, h2, h2, W1, W1, b1.reshape(1, D), W2,
                              b2.reshape(1, D))
    return out


# speedup vs baseline: 2.0314x; 1.0034x over previous
import functools

import jax
import jax.numpy as jnp
from jax import lax
from jax.experimental import pallas as pl
from jax.experimental.pallas import tpu as pltpu
from jax.experimental.pallas import tpu_sc as plsc

_NC = 2
_NS = 16
_L = 16

_EC = 80
_CB = 25
_NB = 5


@functools.lru_cache(maxsize=None)
def _make_aggregate(N, NP, E, Dh):
    n_per_tile = NP // _NS
    e_per_tile = E // _NS
    n_chunks_init = n_per_tile // _EC
    assert n_per_tile % _EC == 0
    assert e_per_tile == _NB * _CB * _EC and _CB % 2 == 1

    mesh = plsc.VectorSubcoreMesh(core_axis_name="c", subcore_axis_name="s")

    def body(xpk_hbm, ea2_hbm, sd_hbm, out_hbm,
             haggr, bufb0, bufb1, bufe0, bufe1,
             idxx0, idxx1, idxe0, idxe1, dstb0, dstb1,
             sd_b,
             semx0, semx1, seme0, seme1):
        k = lax.axis_index("c")
        t = lax.axis_index("s")
        lanes = jnp.arange(_L, dtype=jnp.int32)

        def zrows(i, _):
            for rr in range(4):
                r = i * 4 + rr
                for j in range(Dh // _L):
                    bufe0[r, pl.ds(j * _L, _L)] = jnp.zeros((_L,), jnp.float32)
            return 0

        lax.fori_loop(0, _EC // 4, zrows, 0)

        def init_chunk(c, _):
            base = t * n_per_tile + c * _EC
            pltpu.sync_copy(bufe0, haggr.at[pl.ds(base, _EC)])
            return 0

        lax.fori_loop(0, n_chunks_init, init_chunk, 0)
        plsc.subcore_barrier()

        def build_idx(lc, idxx, dstb):
            for i in range(_EC // _L):
                sl = pl.ds(i * _L, _L)
                idxx[sl] = sd_b[0, pl.ds(lc * _EC + i * _L, _L)] * 2 + k
                dstb[sl] = sd_b[1, pl.ds(lc * _EC + i * _L, _L)]

        def issue_gather(e0, idxx, idxe, bufb, bufe, semx, seme):
            pltpu.async_copy(xpk_hbm.at[idxx], bufb, semx)
            for i in range(_EC // _L):
                idxe[pl.ds(i * _L, _L)] = (e0 + i * _L + lanes) * 2 + k
            pltpu.async_copy(ea2_hbm.at[idxe], bufe, seme)

        def wait_gather(idxx, idxe, bufb, bufe, semx, seme):
            pltpu.make_async_copy(xpk_hbm.at[idxx], bufb, semx).wait()
            pltpu.make_async_copy(ea2_hbm.at[idxe], bufe, seme).wait()

        def compute(bufb, bufe):
            def rows(i, _):
                for rr in range(4):
                    r = i * 4 + rr
                    for j in range(Dh // _L):
                        sl = pl.ds(j * _L, _L)
                        bufe[r, sl] = jnp.maximum(bufb[r, sl] + bufe[r, sl], 0.0)
                return 0

            lax.fori_loop(0, _EC // 4, rows, 0)

        def scatter(bufe, dstb):
            pltpu.sync_copy(bufe, haggr.at[dstb], add=True)

        def batch(b, _):
            pltpu.sync_copy(sd_hbm.at[t, b], sd_b)
            e_base = (t * _NB + b) * _CB * _EC

            build_idx(0, idxx0, dstb0)
            issue_gather(e_base, idxx0, idxe0, bufb0, bufe0, semx0, seme0)

            def pair(j, _):
                c0 = j * 2
                e1 = e_base + (c0 + 1) * _EC
                e2 = e_base + (c0 + 2) * _EC
                build_idx(c0 + 1, idxx1, dstb1)
                issue_gather(e1, idxx1, idxe1, bufb1, bufe1, semx1, seme1)
                wait_gather(idxx0, idxe0, bufb0, bufe0, semx0, seme0)
                compute(bufb0, bufe0)
                scatter(bufe0, dstb0)
                build_idx(c0 + 2, idxx0, dstb0)
                issue_gather(e2, idxx0, idxe0, bufb0, bufe0, semx0, seme0)
                wait_gather(idxx1, idxe1, bufb1, bufe1, semx1, seme1)
                compute(bufb1, bufe1)
                scatter(bufe1, dstb1)
                return 0

            lax.fori_loop(0, (_CB - 1) // 2, pair, 0)

            wait_gather(idxx0, idxe0, bufb0, bufe0, semx0, seme0)
            compute(bufb0, bufe0)
            scatter(bufe0, dstb0)
            return 0

        lax.fori_loop(0, _NB, batch, 0)
        plsc.subcore_barrier()

        base = t * n_per_tile
        pltpu.sync_copy(haggr.at[pl.ds(base, n_per_tile)],
                        out_hbm.at[k, pl.ds(base, n_per_tile)])

    return pl.kernel(
        body,
        out_type=jax.ShapeDtypeStruct((_NC, NP, Dh), jnp.float32),
        mesh=mesh,
        scratch_types=[
            pltpu.VMEM_SHARED((NP, Dh), jnp.float32),
            pltpu.VMEM((_EC, Dh), jnp.float32),
            pltpu.VMEM((_EC, Dh), jnp.float32),
            pltpu.VMEM((_EC, Dh), jnp.float32),
            pltpu.VMEM((_EC, Dh), jnp.float32),
            pltpu.VMEM((_EC,), jnp.int32),
            pltpu.VMEM((_EC,), jnp.int32),
            pltpu.VMEM((_EC,), jnp.int32),
            pltpu.VMEM((_EC,), jnp.int32),
            pltpu.VMEM((_EC,), jnp.int32),
            pltpu.VMEM((_EC,), jnp.int32),
            pltpu.VMEM((2, _CB * _EC), jnp.int32),
            pltpu.SemaphoreType.DMA,
            pltpu.SemaphoreType.DMA,
            pltpu.SemaphoreType.DMA,
            pltpu.SemaphoreType.DMA,
        ],
    )


def _mlp_body(x0_ref, x1_ref, h0_ref, h1_ref, w1a_ref, w1b_ref, b1_ref,
              w2_ref, b2_ref, out_ref):
    a0 = x0_ref[...] + h0_ref[0]
    a1 = x1_ref[...] + h1_ref[0]
    tm = jnp.dot(a0, w1a_ref[...], preferred_element_type=jnp.float32)
    tm = tm + jnp.dot(a1, w1b_ref[...], preferred_element_type=jnp.float32)
    tm = jnp.maximum(tm + b1_ref[...], 0.0)
    out_ref[...] = jnp.dot(tm, w2_ref[...], preferred_element_type=jnp.float32) + b2_ref[...]


@functools.lru_cache(maxsize=None)
def _make_mlp(N, NP, D, R=1000):
    Dh = D // 2
    grid = (N // R,)
    return pl.pallas_call(
        _mlp_body,
        grid=grid,
        in_specs=[
            pl.BlockSpec((R, Dh), lambda i: (i, 0)),
            pl.BlockSpec((R, Dh), lambda i: (i, 1)),
            pl.BlockSpec((1, R, Dh), lambda i: (0, i, 0)),
            pl.BlockSpec((1, R, Dh), lambda i: (1, i, 0)),
            pl.BlockSpec((Dh, D), lambda i: (0, 0)),
            pl.BlockSpec((Dh, D), lambda i: (1, 0)),
            pl.BlockSpec((1, D), lambda i: (0, 0)),
            pl.BlockSpec((D, D), lambda i: (0, 0)),
            pl.BlockSpec((1, D), lambda i: (0, 0)),
        ],
        out_specs=pl.BlockSpec((R, D), lambda i: (i, 0)),
        out_shape=jax.ShapeDtypeStruct((N, D), jnp.float32),
    )


def kernel(x, edge_index, edge_attr, W1, b1, W2, b2):
    N, D = x.shape
    E = edge_attr.shape[0]
    Dh = D // 2
    NP = ((N + _NS * _EC - 1) // (_NS * _EC)) * (_NS * _EC)
    x2 = x.reshape(N * 2, Dh)
    ea2 = edge_attr.reshape(E * 2, Dh)
    sd = edge_index.reshape(2, _NS, _NB, _CB * _EC).transpose(1, 2, 0, 3)
    h2 = _make_aggregate(N, NP, E, Dh)(x2, ea2, sd)
    out = _make_mlp(N, NP, D)(x, x, h2, h2, W1, W1, b1.reshape(1, D), W2,
                              b2.reshape(1, D))
    return out
